# baseline (speedup 1.0000x reference)
import functools

import jax
import jax.numpy as jnp
from jax import lax
from jax.experimental import pallas as pl
from jax.experimental.pallas import tpu as pltpu
from jax.experimental.pallas import tpu_sc as plsc

N_NODES = 50000
ATOM_DIM = 32
BOND_DIM = 16
N_EDGES = 100000
QUARTER = N_EDGES // 4
Q_ROWS = QUARTER

NC = 2
NS = 16
NW = NC * NS

W_CHUNK = 3128
W_TAIL = QUARTER - 7 * W_CHUNK

TC_BLOCK4 = 1024
TC_GRID = (Q_ROWS + TC_BLOCK4 - 1) // TC_BLOCK4

ACC_ROWS = 50048
ZERO_ROWS = ACC_ROWS // NS
OUT_ROWS = N_NODES // NS
HALF = ATOM_DIM // 2


def _gather_body(atom_hbm, cols_hbm, out_hbm, idx_v, rows_v, sem):
    wid = lax.axis_index("s") * NC + lax.axis_index("c")
    q = wid // 8
    j = wid % 8
    base_e = q * QUARTER + j * W_CHUNK
    base_r = j * W_CHUNK

    def go(size):
        pltpu.sync_copy(cols_hbm.at[pl.ds(N_EDGES + base_e, size)],
                        idx_v.at[pl.ds(0, size)])
        pltpu.async_copy(atom_hbm.at[idx_v.at[pl.ds(0, size)]],
                         rows_v.at[pl.ds(0, size)], sem).wait()
        pltpu.sync_copy(rows_v.at[pl.ds(0, size)],
                        out_hbm.at[pl.ds(base_r, size),
                                   pl.ds(q * ATOM_DIM, ATOM_DIM)])

    @pl.when(j < 7)
    def _():
        go(W_CHUNK)

    @pl.when(j == 7)
    def _():
        go(W_TAIL)


def _sc_gather(atom_features, cols_flat):
    mesh = plsc.VectorSubcoreMesh(core_axis_name="c", subcore_axis_name="s")
    k = functools.partial(
        pl.kernel,
        mesh=mesh,
        out_type=jax.ShapeDtypeStruct((Q_ROWS, 128), jnp.float32),
        scratch_types=[
            pltpu.VMEM((W_CHUNK,), jnp.int32),
            pltpu.VMEM((W_CHUNK, ATOM_DIM), jnp.float32),
            pltpu.SemaphoreType.DMA,
        ],
        compiler_params=pltpu.CompilerParams(use_tc_tiling_on_sc=False),
    )(_gather_body)
    return k(atom_features, cols_flat)


def _tc_body(bond_ref, nbr_ref, wbig_ref, rbig_ref, fbig_ref, bbig_ref,
             out_ref):
    bond4 = bond_ref[...]
    nbr4 = nbr_ref[...]
    bond_rep = jnp.dot(bond4, rbig_ref[...],
                       preferred_element_type=jnp.float32)
    g = jnp.dot(nbr4, wbig_ref[...], preferred_element_type=jnp.float32)
    tr = jnp.dot(bond_rep * g, fbig_ref[...],
                 preferred_element_type=jnp.float32)
    tr = tr + jnp.dot(nbr4, bbig_ref[...], preferred_element_type=jnp.float32)
    out_ref[...] = tr


def _tc_transform(bond4, nbr4, wbig, rbig, fbig, bbig):
    return pl.pallas_call(
        _tc_body,
        grid=(TC_GRID,),
        in_specs=[
            pl.BlockSpec((TC_BLOCK4, 64), lambda i: (i, 0)),
            pl.BlockSpec((TC_BLOCK4, 128), lambda i: (i, 0)),
            pl.BlockSpec((128, 2048), lambda i: (0, 0)),
            pl.BlockSpec((64, 2048), lambda i: (0, 0)),
            pl.BlockSpec((2048, 128), lambda i: (0, 0)),
            pl.BlockSpec((128, 128), lambda i: (0, 0)),
        ],
        out_specs=pl.BlockSpec((TC_BLOCK4, 128), lambda i: (i, 0)),
        out_shape=jax.ShapeDtypeStruct((Q_ROWS, 128), jnp.float32),
    )(bond4, nbr4, wbig, rbig, fbig, bbig)


SUB_A = 1568
SUB_B_FULL = W_CHUNK - SUB_A
SUB_B_TAIL = W_TAIL - SUB_A


def _scatter_body(cols_hbm, t_hbm, zeros_hbm, out_hbm, acc,
                  idx_a, idx_b, rows_a, rows_b, sem_ia, sem_ib, sem_ra,
                  sem_rb):
    cid = lax.axis_index("c")
    sid = lax.axis_index("s")
    j = sid % 8
    lane0 = cid * HALF

    bufs = [(idx_a, rows_a, sem_ia, sem_ra), (idx_b, rows_b, sem_ib, sem_rb)]

    def go(sub_b):
        subs = []
        for step in range(2):
            c = sid + NS * step
            q = c // 8
            base_e = q * QUARTER + j * W_CHUNK
            base_r = j * W_CHUNK
            qlane = q * ATOM_DIM + lane0
            subs.append((base_e, base_r, qlane, SUB_A))
            subs.append((base_e + SUB_A, base_r + SUB_A, qlane, sub_b))

        def fetch(i, b):
            base_e, base_r, qlane, size = subs[i]
            idx_v, rows_v, sem_i, sem_r = bufs[b]
            di = pltpu.async_copy(cols_hbm.at[pl.ds(base_e, size)],
                                  idx_v.at[pl.ds(0, size)], sem_i)
            dr = pltpu.async_copy(
                t_hbm.at[pl.ds(base_r, size), pl.ds(qlane, HALF)],
                rows_v.at[pl.ds(0, size)], sem_r)
            return di, dr

        pending = fetch(0, 0)
        pltpu.sync_copy(zeros_hbm, acc.at[pl.ds(sid * ZERO_ROWS, ZERO_ROWS)])
        plsc.subcore_barrier()
        for i in range(4):
            nxt = fetch(i + 1, (i + 1) % 2) if i < 3 else None
            di, dr = pending
            di.wait()
            dr.wait()
            size = subs[i][3]
            idx_v, rows_v, _, _ = bufs[i % 2]
            pltpu.sync_copy(rows_v.at[pl.ds(0, size)],
                            acc.at[idx_v.at[pl.ds(0, size)]], add=True)
            pending = nxt

    @pl.when(j < 7)
    def _():
        go(SUB_B_FULL)

    @pl.when(j == 7)
    def _():
        go(SUB_B_TAIL)

    plsc.subcore_barrier()
    obase = sid * OUT_ROWS
    pltpu.sync_copy(acc.at[pl.ds(obase, OUT_ROWS)],
                    out_hbm.at[pl.ds(obase, OUT_ROWS), pl.ds(cid * HALF, HALF)])


def _sc_scatter(cols_flat, t4, zeros_block):
    mesh = plsc.VectorSubcoreMesh(core_axis_name="c", subcore_axis_name="s")
    k = functools.partial(
        pl.kernel,
        mesh=mesh,
        out_type=jax.ShapeDtypeStruct((N_NODES, ATOM_DIM), jnp.float32),
        scratch_types=[
            pltpu.VMEM_SHARED((ACC_ROWS, HALF), jnp.float32),
            pltpu.VMEM((SUB_A,), jnp.int32),
            pltpu.VMEM((SUB_A,), jnp.int32),
            pltpu.VMEM((SUB_A, HALF), jnp.float32),
            pltpu.VMEM((SUB_A, HALF), jnp.float32),
            pltpu.SemaphoreType.DMA,
            pltpu.SemaphoreType.DMA,
            pltpu.SemaphoreType.DMA,
            pltpu.SemaphoreType.DMA,
        ],
        compiler_params=pltpu.CompilerParams(use_tc_tiling_on_sc=False),
    )(_scatter_body)
    return k(cols_flat, t4, zeros_block)


def kernel(atom_features, bond_features, pair_indices, kernel, bias):
    cols_flat = pair_indices.T.reshape(2 * N_EDGES)
    kdim = BOND_DIM * ATOM_DIM
    wt2 = kernel.reshape(BOND_DIM, ATOM_DIM, ATOM_DIM).transpose(2, 0, 1)
    wt2 = wt2.reshape(ATOM_DIM, kdim)
    b2t = bias.reshape(ATOM_DIM, ATOM_DIM).T
    c_ids = jnp.arange(kdim, dtype=jnp.int32)
    r = (c_ids[None, :] // ATOM_DIM
         == jnp.arange(BOND_DIM, dtype=jnp.int32)[:, None]).astype(jnp.float32)
    f = (c_ids[:, None] % ATOM_DIM
         == jnp.arange(ATOM_DIM, dtype=jnp.int32)[None, :]).astype(jnp.float32)
    eye4 = jnp.eye(4, dtype=jnp.float32)
    wbig = jnp.kron(eye4, wt2)
    rbig = jnp.kron(eye4, r)
    fbig = jnp.kron(eye4, f)
    bbig = jnp.kron(eye4, b2t)
    bond4 = bond_features.reshape(4, QUARTER, BOND_DIM)
    bond4 = bond4.transpose(1, 0, 2).reshape(QUARTER, 4 * BOND_DIM)
    zeros_block = jnp.zeros((ZERO_ROWS, HALF), jnp.float32)

    nbr4 = _sc_gather(atom_features, cols_flat)
    t4 = _tc_transform(bond4, nbr4, wbig, rbig, fbig, bbig)
    return _sc_scatter(cols_flat, t4, zeros_block)
